# baseline (speedup 1.0000x reference)
import functools

import jax
import jax.numpy as jnp
from jax import lax
from jax.experimental import pallas as pl
from jax.experimental.pallas import tpu as pltpu
from jax.experimental.pallas import tpu_sc as plsc

_NC, _NS, _L = 2, 16, 16
_NW = _NC * _NS
_MAX_SLIP = 0.4
_MAX_GUESS = 0.4
_T_INV = 1.0 / 50.0
_SW = 8


def _sigmoid(x):
    return 1.0 / (1.0 + jnp.exp(-x))


def _make_sc_kernel(B, H, S, C):
    RPW = B // _NW
    NCHUNK = RPW // C
    GRP = C // _L
    HB = H // _L
    OFF_W1 = 3 * H
    OFF_B1 = OFF_W1 + 48
    OFF_W2 = OFF_B1 + 16
    OFF_B2 = OFF_W2 + 16
    WPACK = OFF_B2 + 16

    mesh = plsc.VectorSubcoreMesh(core_axis_name="c", subcore_axis_name="s",
                                  num_cores=_NC, num_subcores=_NS)

    @functools.partial(
        pl.kernel,
        out_type=jax.ShapeDtypeStruct((B,), jnp.float32),
        mesh=mesh,
        scratch_types=[
            pltpu.VMEM((C,), jnp.int32),
            pltpu.VMEM((C,), jnp.int32),
            pltpu.VMEM((C, H), jnp.float32),
            pltpu.VMEM((C, S, H), jnp.float32),
            pltpu.VMEM((C, H), jnp.float32),
            pltpu.VMEM((C, _SW), jnp.float32),
            pltpu.VMEM((C, _SW), jnp.float32),
            pltpu.VMEM((WPACK,), jnp.float32),
            pltpu.VMEM((C,), jnp.float32),
            pltpu.SemaphoreType.DMA,
        ],
        compiler_params=pltpu.CompilerParams(
            needs_layout_passes=False, use_tc_tiling_on_sc=False),
    )
    def k(user, item, knowledge, theta_t, us_t, it_t, q_t,
          wpack, out, idx_u, idx_i, th_v, q_v, kn_v, us_v, it_v,
          wp_v, out_v, sem):
        wid = lax.axis_index("s") * _NC + lax.axis_index("c")
        pltpu.sync_copy(wpack, wp_v)
        lanes = lax.iota(jnp.int32, _L)
        z16 = jnp.zeros((_L,), jnp.int32)
        o16 = jnp.ones((_L,), jnp.int32)
        w1c = [wp_v[pl.ds(OFF_W1 + 16 * kk, 16)] for kk in range(3)]
        b1v = wp_v[pl.ds(OFF_B1, 16)]
        w2v = wp_v[pl.ds(OFF_W2, 16)]
        b2s = wp_v[pl.ds(OFF_B2, 16)][0]

        for chunk in range(NCHUNK):
            base = wid * RPW + chunk * C
            pltpu.sync_copy(user.at[pl.ds(base, C)], idx_u)
            pltpu.sync_copy(item.at[pl.ds(base, C)], idx_i)
            cps = (
                pltpu.async_copy(theta_t.at[idx_u], th_v, sem),
                pltpu.async_copy(q_t.at[idx_i], q_v, sem),
                pltpu.async_copy(knowledge.at[pl.ds(base, C)], kn_v, sem),
                pltpu.async_copy(us_t.at[idx_u], us_v, sem),
                pltpu.async_copy(it_t.at[idx_i], it_v, sem),
            )
            for cp in cps:
                cp.wait()

            def group(g, _):
                r0 = g * _L
                ridx = r0 + lanes
                a0 = plsc.load_gather(us_v, [ridx, z16])
                a1 = plsc.load_gather(us_v, [ridx, o16])
                a2 = plsc.load_gather(us_v, [ridx, z16 + 2])
                acc = jnp.zeros((_L,), jnp.float32)
                for j in range(16):
                    hj = w1c[0][j] * a0 + w1c[1][j] * a1 + w1c[2][j] * a2 + b1v[j]
                    acc = acc + w2v[j] * jnp.maximum(hj, 0.0)
                af = _sigmoid(acc + b2s)
                f0 = af * a0
                f1 = af * a1
                f2 = af * a2

                def hblock(hb, carry):
                    m0, m1 = carry
                    h0 = hb * _L
                    aw0b = wp_v[pl.ds(h0, 16)]
                    aw1b = wp_v[pl.ds(H + h0, 16)]
                    aw2b = wp_v[pl.ds(2 * H + h0, 16)]
                    for j in range(_L):
                        hvec = jnp.full((_L,), h0 + j, jnp.int32)
                        gt = plsc.load_gather(th_v, [ridx, hvec])
                        th = gt + f0 * aw0b[j] + f1 * aw1b[j] + f2 * aw2b[j]
                        sg = _sigmoid(th) - 0.5
                        kk = plsc.load_gather(kn_v, [ridx, hvec]) * sg
                        q0 = _sigmoid(plsc.load_gather(q_v, [ridx, z16, hvec]))
                        q1 = _sigmoid(plsc.load_gather(q_v, [ridx, o16, hvec]))
                        m0 = m0 + kk * q0
                        m1 = m1 + kk * q1
                    return m0, m1

                zero = jnp.zeros((_L,), jnp.float32)
                m0, m1 = lax.fori_loop(0, HB, hblock, (zero, zero))
                p0 = _sigmoid(m0 * _T_INV)
                p1 = _sigmoid(m1 * _T_INV)
                slip = _sigmoid(plsc.load_gather(it_v, [ridx, z16])) * _MAX_SLIP
                guess = _sigmoid(plsc.load_gather(it_v, [ridx, o16])) * _MAX_GUESS
                w0 = plsc.load_gather(it_v, [ridx, z16 + 2])
                w1 = plsc.load_gather(it_v, [ridx, z16 + 3])
                sp0 = _sigmoid(w0 - w1)
                span = 1.0 - slip - guess
                c0 = guess + span * p0
                c1 = guess + span * p1
                out_v[pl.ds(r0, _L)] = c1 + sp0 * (c0 - c1)
                return 0

            lax.fori_loop(0, GRP, group, 0)
            pltpu.sync_copy(out_v, out.at[pl.ds(base, C)])

    return k


def kernel(user, item, knowledge, theta_table, affect_table, slip_table,
           guess_table, strategy_weights, strategy_q, affect_weight,
           W1, b1, W2, b2):
    B = user.shape[0]
    H = theta_table.shape[1]
    S = strategy_weights.shape[1]
    U = affect_table.shape[0]
    I = slip_table.shape[0]
    us_t = jnp.concatenate(
        [affect_table, jnp.zeros((U, _SW - 3), jnp.float32)], axis=1)
    it_t = jnp.concatenate(
        [slip_table, guess_table, strategy_weights,
         jnp.zeros((I, _SW - 2 - S), jnp.float32)], axis=1)
    wpack = jnp.concatenate([
        affect_weight.reshape(-1),
        W1.T.reshape(-1),
        b1.reshape(-1),
        W2.reshape(-1),
        b2.reshape(-1),
        jnp.zeros((15,), jnp.float32),
    ])
    k = _make_sc_kernel(B, H, S, C=128)
    return k(user.astype(jnp.int32), item.astype(jnp.int32), knowledge,
             theta_table, us_t, it_t, strategy_q, wpack)
